# Initial kernel scaffold; baseline (speedup 1.0000x reference)
import jax
import jax.numpy as jnp
from jax.experimental import pallas as pl


def kernel(verts, graph_pos, processed, W1, b1, W2, b2):
    raise NotImplementedError("write your pallas kernel here")


# speedup vs baseline: 1.4805x; 1.4805x over previous
import functools

import jax
import jax.numpy as jnp
from jax import lax
from jax.experimental import pallas as pl
from jax.experimental.pallas import tpu as pltpu
from jax.experimental.pallas import tpu_sc as plsc


def _mlp_body(p_ref, w1_ref, b1_ref, w2_ref, b2_ref, y_ref):
    h = jnp.dot(p_ref[...], w1_ref[...], preferred_element_type=jnp.float32)
    h = jnp.maximum(h + b1_ref[...], 0.0)
    y_ref[...] = (
        jnp.dot(h, w2_ref[...], preferred_element_type=jnp.float32) + b2_ref[...]
    )


def _mlp(processed, W1, b1, W2, b2, block_n=1024):
    n, c = processed.shape
    grid = (n // block_n,)
    return pl.pallas_call(
        _mlp_body,
        grid=grid,
        in_specs=[
            pl.BlockSpec((block_n, c), lambda i: (i, 0)),
            pl.BlockSpec((c, c), lambda i: (0, 0)),
            pl.BlockSpec((1, c), lambda i: (0, 0)),
            pl.BlockSpec((c, 1), lambda i: (0, 0)),
            pl.BlockSpec((1, 1), lambda i: (0, 0)),
        ],
        out_specs=pl.BlockSpec((block_n, 1), lambda i: (i, 0)),
        out_shape=jax.ShapeDtypeStruct((n, 1), jnp.float32),
    )(processed, W1, b1.reshape(1, c), W2, b2.reshape(1, 1))


def _argmin_body(q_ref, g_ref, v2_ref, g2_ref, idx_ref, *, n):
    m = jnp.dot(
        q_ref[...].astype(jnp.bfloat16),
        g_ref[...].astype(jnp.bfloat16),
        preferred_element_type=jnp.float32,
    )
    s = (v2_ref[...] + g2_ref[...]) - 2.0 * m
    mn = jnp.min(s, axis=1, keepdims=True)
    iota = lax.broadcasted_iota(jnp.int32, s.shape, 1)
    idx_ref[...] = jnp.min(
        jnp.where(s == mn, iota, jnp.int32(n)), axis=1, keepdims=True
    )


def _argmin(q, g_mat, v2, g2, block_v=512):
    v = q.shape[0]
    n = g_mat.shape[1]
    grid = (v // block_v,)
    return pl.pallas_call(
        functools.partial(_argmin_body, n=n),
        grid=grid,
        in_specs=[
            pl.BlockSpec((block_v, 8), lambda i: (i, 0)),
            pl.BlockSpec((8, n), lambda i: (0, 0)),
            pl.BlockSpec((block_v, 1), lambda i: (i, 0)),
            pl.BlockSpec((1, n), lambda i: (0, 0)),
        ],
        out_specs=pl.BlockSpec((block_v, 1), lambda i: (i, 0)),
        out_shape=jax.ShapeDtypeStruct((v, 1), jnp.int32),
    )(q, g_mat, v2, g2)


def _make_sc_gather(v, n):
    nw = 32
    b_per_w = v // nw
    mesh = plsc.VectorSubcoreMesh(core_axis_name="c", subcore_axis_name="s")

    @functools.partial(
        pl.kernel,
        mesh=mesh,
        out_type=jax.ShapeDtypeStruct((v,), jnp.float32),
        scratch_types=[
            pltpu.VMEM((b_per_w,), jnp.int32),
            pltpu.VMEM((b_per_w,), jnp.float32),
            pltpu.SemaphoreType.DMA,
        ],
    )
    def gather_kernel(y_hbm, idx_hbm, out_hbm, idx_v, out_v, sem):
        wid = lax.axis_index("s") * 2 + lax.axis_index("c")
        base = wid * b_per_w
        pltpu.sync_copy(idx_hbm.at[pl.ds(base, b_per_w)], idx_v)
        pltpu.async_copy(y_hbm.at[idx_v], out_v, sem).wait()
        pltpu.sync_copy(out_v, out_hbm.at[pl.ds(base, b_per_w)])

    return gather_kernel


def kernel(verts, graph_pos, processed, W1, b1, W2, b2):
    v = verts.shape[0]
    n, c = processed.shape

    q = jnp.concatenate([verts, jnp.zeros((v, 5), jnp.float32)], axis=1)
    g_mat = jnp.concatenate(
        [graph_pos.T, jnp.zeros((5, n), jnp.float32)], axis=0)
    v2 = jnp.sum(verts * verts, axis=1, keepdims=True)
    g2 = jnp.sum(graph_pos * graph_pos, axis=1)[None, :]

    y = _mlp(processed, W1, b1, W2, b2)
    idx = _argmin(q, g_mat, v2, g2)
    out = _make_sc_gather(v, n)(y.reshape(n), idx.reshape(v))
    return out.reshape(v, 1)
